# baseline (speedup 1.0000x reference)
import functools

import jax
import jax.numpy as jnp
from jax import lax
from jax.experimental import pallas as pl
from jax.experimental.pallas import tpu as pltpu
from jax.experimental.pallas import tpu_sc as plsc

B = 16384
D_ARTIST, D_ALBUM, D_GENRE = 64, 32, 16
H1, H2 = 256, 128

NC, NS = 2, 16
NW = NC * NS
BPW = B // NW
WIN = 128
NWINDOW = BPW // WIN

_sc_mesh = plsc.VectorSubcoreMesh(core_axis_name="c", subcore_axis_name="s")


@functools.partial(
    pl.kernel,
    out_type=[
        jax.ShapeDtypeStruct((B, 128), jnp.float32),
        jax.ShapeDtypeStruct((B, 128), jnp.float32),
        jax.ShapeDtypeStruct((B, 128), jnp.float32),
    ],
    mesh=_sc_mesh,
    scratch_types=[
        pltpu.VMEM((1, WIN), jnp.int32),
        pltpu.VMEM((1, WIN), jnp.int32),
        pltpu.VMEM((1, WIN), jnp.int32),
        pltpu.VMEM((WIN, 128), jnp.float32),
        pltpu.VMEM((WIN, 128), jnp.float32),
        pltpu.VMEM((WIN, 128), jnp.float32),
        pltpu.SemaphoreType.DMA,
        pltpu.SemaphoreType.DMA,
        pltpu.SemaphoreType.DMA,
    ],
)
def _sc_gather(aidx_h, bidx_h, gidx_h, ap_h, bp_h, gp_h,
               oa_h, ob_h, og_h,
               ixa, ixb, ixg, ra, rb, rg, sema, semb, semg):
    wid = lax.axis_index("s") * NC + lax.axis_index("c")

    for j in range(NWINDOW):
        r = wid * NWINDOW + j
        pltpu.sync_copy(aidx_h.at[pl.ds(r, 1)], ixa)
        pltpu.sync_copy(bidx_h.at[pl.ds(r, 1)], ixb)
        pltpu.sync_copy(gidx_h.at[pl.ds(r, 1)], ixg)
        ca = pltpu.async_copy(ap_h.at[ixa.at[0]], ra, sema)
        cb = pltpu.async_copy(bp_h.at[ixb.at[0]], rb, semb)
        cg = pltpu.async_copy(gp_h.at[ixg.at[0]], rg, semg)
        ca.wait()
        cb.wait()
        cg.wait()
        base = wid * BPW + j * WIN
        pltpu.sync_copy(ra, oa_h.at[pl.ds(base, WIN)])
        pltpu.sync_copy(rb, ob_h.at[pl.ds(base, WIN)])
        pltpu.sync_copy(rg, og_h.at[pl.ds(base, WIN)])


def _repack_body(*refs):
    o_ref = refs[-1]
    o_ref[...] = jnp.concatenate([r[...] for r in refs[:-1]], axis=1)


def _repack(table, d, rb):
    v = table.shape[0]
    k = 128 // d
    vk = v // k
    specs = [
        pl.BlockSpec((rb, d), functools.partial(lambda j, i: (i + j * (vk // rb), 0), j))
        for j in range(k)
    ]
    return pl.pallas_call(
        _repack_body,
        grid=(vk // rb,),
        in_specs=specs,
        out_specs=pl.BlockSpec((rb, 128), lambda i: (i, 0)),
        out_shape=jax.ShapeDtypeStruct((vk, 128), jnp.float32),
    )(*([table] * k))


BLK = 2048


def _subselect(packed, sel, d):
    out = jnp.zeros((BLK, d), jnp.float32)
    for grp in range(128 // d):
        m = (sel == grp).astype(jnp.float32)
        out += m * packed[:, grp * d:(grp + 1) * d]
    return out


def _mlp_body(num_ref, ea_ref, eb_ref, eg_ref, aid_ref, bid_ref, gid_ref,
              w1n_ref, w1a_ref, w1b_ref, w1g_ref, b1_ref, w2_ref, b2_ref,
              o_ref):
    ea = _subselect(ea_ref[...], aid_ref[...], D_ARTIST)
    eb = _subselect(eb_ref[...], bid_ref[...], D_ALBUM)
    eg = _subselect(eg_ref[...], gid_ref[...], D_GENRE)
    h = jnp.dot(num_ref[...], w1n_ref[...], preferred_element_type=jnp.float32)
    h += jnp.dot(ea, w1a_ref[...], preferred_element_type=jnp.float32)
    h += jnp.dot(eb, w1b_ref[...], preferred_element_type=jnp.float32)
    h += jnp.dot(eg, w1g_ref[...], preferred_element_type=jnp.float32)
    h = jnp.maximum(h + b1_ref[...], 0.0)
    o = jnp.dot(h, w2_ref[...], preferred_element_type=jnp.float32)
    o_ref[...] = jnp.maximum(o + b2_ref[...], 0.0)


def _mlp(num, ea, eb, eg, aid, bid, gid, w1n, w1a, w1b, w1g, b1, w2, b2):
    grid = (B // BLK,)
    full = lambda shape: pl.BlockSpec(shape, lambda i: (0, 0))
    blk2 = lambda shape: pl.BlockSpec(shape, lambda i: (i, 0))
    return pl.pallas_call(
        _mlp_body,
        grid=grid,
        in_specs=[
            blk2((BLK, 9)),
            blk2((BLK, 128)),
            blk2((BLK, 128)),
            blk2((BLK, 128)),
            blk2((BLK, 1)),
            blk2((BLK, 1)),
            blk2((BLK, 1)),
            full((9, H1)),
            full((D_ARTIST, H1)),
            full((D_ALBUM, H1)),
            full((D_GENRE, H1)),
            full((1, H1)),
            full((H1, H2)),
            full((1, H2)),
        ],
        out_specs=blk2((BLK, H2)),
        out_shape=jax.ShapeDtypeStruct((B, H2), jnp.float32),
    )(num, ea, eb, eg, aid, bid, gid, w1n, w1a, w1b, w1g, b1, w2, b2)


def kernel(danceability, energy, loudness, speechiness, acousticness,
           instrumentalness, liveness, valence, tempo,
           artist_id, album_id, genre_id,
           E_artist, E_album, E_genre, W1, b1, W2, b2):
    ap = _repack(E_artist, D_ARTIST, 4000)
    bp = _repack(E_album, D_ALBUM, 2000)
    gp = E_genre.reshape(125, 128)
    aidx = (artist_id % 500000).reshape(NW * NWINDOW, WIN)
    bidx = (album_id % 250000).reshape(NW * NWINDOW, WIN)
    gidx = (genre_id >> 3).reshape(NW * NWINDOW, WIN)
    ea, eb, eg = _sc_gather(aidx, bidx, gidx, ap, bp, gp)
    num = jnp.stack([danceability, energy, loudness, speechiness, acousticness,
                     instrumentalness, liveness, valence, tempo], axis=1)
    return _mlp(num, ea, eb, eg,
                (artist_id // 500000).reshape(B, 1),
                (album_id // 250000).reshape(B, 1),
                (genre_id & 7).reshape(B, 1),
                W1[:9], W1[9:9 + D_ARTIST],
                W1[9 + D_ARTIST:9 + D_ARTIST + D_ALBUM],
                W1[9 + D_ARTIST + D_ALBUM:],
                b1.reshape(1, H1), W2, b2.reshape(1, H2))


# speedup vs baseline: 1.7282x; 1.7282x over previous
import functools

import jax
import jax.numpy as jnp
from jax import lax
from jax.experimental import pallas as pl
from jax.experimental.pallas import tpu as pltpu
from jax.experimental.pallas import tpu_sc as plsc

B = 16384
D_ARTIST, D_ALBUM, D_GENRE = 64, 32, 16
H1, H2 = 256, 128

NC, NS = 2, 16
NW = NC * NS
BPW = B // NW
WIN = 128
NWINDOW = BPW // WIN

_sc_mesh = plsc.VectorSubcoreMesh(core_axis_name="c", subcore_axis_name="s")


@functools.partial(
    pl.kernel,
    out_type=[
        jax.ShapeDtypeStruct((B, 128), jnp.float32),
        jax.ShapeDtypeStruct((B, 128), jnp.float32),
        jax.ShapeDtypeStruct((B, 128), jnp.float32),
    ],
    mesh=_sc_mesh,
    scratch_types=[
        pltpu.VMEM((1, WIN), jnp.int32),
        pltpu.VMEM((1, WIN), jnp.int32),
        pltpu.VMEM((1, WIN), jnp.int32),
        pltpu.VMEM((WIN, 128), jnp.float32),
        pltpu.VMEM((WIN, 128), jnp.float32),
        pltpu.VMEM((WIN, 128), jnp.float32),
        pltpu.SemaphoreType.DMA,
        pltpu.SemaphoreType.DMA,
        pltpu.SemaphoreType.DMA,
    ],
)
def _sc_gather(aidx_h, bidx_h, gidx_h, ap_h, bp_h, gp_h,
               oa_h, ob_h, og_h,
               ixa, ixb, ixg, ra, rb, rg, sema, semb, semg):
    wid = lax.axis_index("s") * NC + lax.axis_index("c")

    for j in range(NWINDOW):
        r = wid * NWINDOW + j
        pltpu.sync_copy(aidx_h.at[pl.ds(r, 1)], ixa)
        pltpu.sync_copy(bidx_h.at[pl.ds(r, 1)], ixb)
        pltpu.sync_copy(gidx_h.at[pl.ds(r, 1)], ixg)
        ca = pltpu.async_copy(ap_h.at[ixa.at[0]], ra, sema)
        cb = pltpu.async_copy(bp_h.at[ixb.at[0]], rb, semb)
        cg = pltpu.async_copy(gp_h.at[ixg.at[0]], rg, semg)
        ca.wait()
        cb.wait()
        cg.wait()
        base = wid * BPW + j * WIN
        pltpu.sync_copy(ra, oa_h.at[pl.ds(base, WIN)])
        pltpu.sync_copy(rb, ob_h.at[pl.ds(base, WIN)])
        pltpu.sync_copy(rg, og_h.at[pl.ds(base, WIN)])


CB = 2048
S_ARTIST = 501760
S_ALBUM = 251904


def _repackT_body(*refs):
    o_ref = refs[-1]
    o_ref[...] = jnp.concatenate([r[...].T for r in refs[:-1]], axis=1)


def _repackT(table_t, d, split):
    v = table_t.shape[1]
    k = 128 // d
    nblk = split // CB
    last = (v + CB - 1) // CB - 1
    specs = [
        pl.BlockSpec(
            (d, CB),
            functools.partial(
                lambda j, i: (0, jnp.minimum(i + j * nblk, last)), j))
        for j in range(k)
    ]
    return pl.pallas_call(
        _repackT_body,
        grid=(nblk,),
        in_specs=specs,
        out_specs=pl.BlockSpec((CB, 128), lambda i: (i, 0)),
        out_shape=jax.ShapeDtypeStruct((split, 128), jnp.float32),
    )(*([table_t] * k))


BLK = 2048


def _subselect(packed, sel, d):
    out = jnp.zeros((BLK, d), jnp.float32)
    for grp in range(128 // d):
        out += jnp.where(sel == grp, packed[:, grp * d:(grp + 1) * d], 0.0)
    return out


def _mlp_body(num_ref, ea_ref, eb_ref, eg_ref, aid_ref, bid_ref, gid_ref,
              w1n_ref, w1a_ref, w1b_ref, w1g_ref, b1_ref, w2_ref, b2_ref,
              o_ref):
    ea = _subselect(ea_ref[...], aid_ref[...], D_ARTIST)
    eb = _subselect(eb_ref[...], bid_ref[...], D_ALBUM)
    eg = _subselect(eg_ref[...], gid_ref[...], D_GENRE)
    h = jnp.dot(num_ref[...], w1n_ref[...], preferred_element_type=jnp.float32)
    h += jnp.dot(ea, w1a_ref[...], preferred_element_type=jnp.float32)
    h += jnp.dot(eb, w1b_ref[...], preferred_element_type=jnp.float32)
    h += jnp.dot(eg, w1g_ref[...], preferred_element_type=jnp.float32)
    h = jnp.maximum(h + b1_ref[...], 0.0)
    o = jnp.dot(h, w2_ref[...], preferred_element_type=jnp.float32)
    o_ref[...] = jnp.maximum(o + b2_ref[...], 0.0)


def _mlp(num, ea, eb, eg, aid, bid, gid, w1n, w1a, w1b, w1g, b1, w2, b2):
    grid = (B // BLK,)
    full = lambda shape: pl.BlockSpec(shape, lambda i: (0, 0))
    blk2 = lambda shape: pl.BlockSpec(shape, lambda i: (i, 0))
    return pl.pallas_call(
        _mlp_body,
        grid=grid,
        in_specs=[
            blk2((BLK, 9)),
            blk2((BLK, 128)),
            blk2((BLK, 128)),
            blk2((BLK, 128)),
            blk2((BLK, 1)),
            blk2((BLK, 1)),
            blk2((BLK, 1)),
            full((9, H1)),
            full((D_ARTIST, H1)),
            full((D_ALBUM, H1)),
            full((D_GENRE, H1)),
            full((1, H1)),
            full((H1, H2)),
            full((1, H2)),
        ],
        out_specs=blk2((BLK, H2)),
        out_shape=jax.ShapeDtypeStruct((B, H2), jnp.float32),
    )(num, ea, eb, eg, aid, bid, gid, w1n, w1a, w1b, w1g, b1, w2, b2)


def kernel(danceability, energy, loudness, speechiness, acousticness,
           instrumentalness, liveness, valence, tempo,
           artist_id, album_id, genre_id,
           E_artist, E_album, E_genre, W1, b1, W2, b2):
    ap = _repackT(E_artist.T, D_ARTIST, S_ARTIST)
    bp = _repackT(E_album.T, D_ALBUM, S_ALBUM)
    gp = E_genre.reshape(125, 128)
    asel = (artist_id >= S_ARTIST).astype(jnp.int32)
    bsel = album_id // S_ALBUM
    aidx = (artist_id - asel * S_ARTIST).reshape(NW * NWINDOW, WIN)
    bidx = (album_id - bsel * S_ALBUM).reshape(NW * NWINDOW, WIN)
    gidx = (genre_id >> 3).reshape(NW * NWINDOW, WIN)
    ea, eb, eg = _sc_gather(aidx, bidx, gidx, ap, bp, gp)
    num = jnp.stack([danceability, energy, loudness, speechiness, acousticness,
                     instrumentalness, liveness, valence, tempo], axis=1)
    return _mlp(num, ea, eb, eg,
                asel.reshape(B, 1),
                bsel.reshape(B, 1),
                (genre_id & 7).reshape(B, 1),
                W1[:9], W1[9:9 + D_ARTIST],
                W1[9 + D_ARTIST:9 + D_ARTIST + D_ALBUM],
                W1[9 + D_ARTIST + D_ALBUM:],
                b1.reshape(1, H1), W2, b2.reshape(1, H2))


# speedup vs baseline: 1.7289x; 1.0004x over previous
import functools

import jax
import jax.numpy as jnp
from jax import lax
from jax.experimental import pallas as pl
from jax.experimental.pallas import tpu as pltpu
from jax.experimental.pallas import tpu_sc as plsc

B = 16384
D_ARTIST, D_ALBUM, D_GENRE = 64, 32, 16
H1, H2 = 256, 128

NC, NS = 2, 16
NW = NC * NS
BPW = B // NW
WIN = 128
NWINDOW = BPW // WIN

_sc_mesh = plsc.VectorSubcoreMesh(core_axis_name="c", subcore_axis_name="s")


@functools.partial(
    pl.kernel,
    out_type=[
        jax.ShapeDtypeStruct((B, 128), jnp.float32),
        jax.ShapeDtypeStruct((B, 128), jnp.float32),
        jax.ShapeDtypeStruct((B, 128), jnp.float32),
    ],
    mesh=_sc_mesh,
    scratch_types=[
        pltpu.VMEM((1, WIN), jnp.int32),
        pltpu.VMEM((1, WIN), jnp.int32),
        pltpu.VMEM((1, WIN), jnp.int32),
        pltpu.VMEM((WIN, 128), jnp.float32),
        pltpu.VMEM((WIN, 128), jnp.float32),
        pltpu.VMEM((WIN, 128), jnp.float32),
        pltpu.SemaphoreType.DMA,
        pltpu.SemaphoreType.DMA,
        pltpu.SemaphoreType.DMA,
    ],
)
def _sc_gather(aidx_h, bidx_h, gidx_h, ap_h, bp_h, gp_h,
               oa_h, ob_h, og_h,
               ixa, ixb, ixg, ra, rb, rg, sema, semb, semg):
    wid = lax.axis_index("s") * NC + lax.axis_index("c")

    for j in range(NWINDOW):
        r = wid * NWINDOW + j
        pltpu.sync_copy(aidx_h.at[pl.ds(r, 1)], ixa)
        pltpu.sync_copy(bidx_h.at[pl.ds(r, 1)], ixb)
        pltpu.sync_copy(gidx_h.at[pl.ds(r, 1)], ixg)
        ca = pltpu.async_copy(ap_h.at[ixa.at[0]], ra, sema)
        cb = pltpu.async_copy(bp_h.at[ixb.at[0]], rb, semb)
        cg = pltpu.async_copy(gp_h.at[ixg.at[0]], rg, semg)
        ca.wait()
        cb.wait()
        cg.wait()
        base = wid * BPW + j * WIN
        pltpu.sync_copy(ra, oa_h.at[pl.ds(base, WIN)])
        pltpu.sync_copy(rb, ob_h.at[pl.ds(base, WIN)])
        pltpu.sync_copy(rg, og_h.at[pl.ds(base, WIN)])


CB = 2048
S_ARTIST = 501760
S_ALBUM = 251904


def _repackT_body(*refs):
    o_ref = refs[-1]
    o_ref[...] = jnp.concatenate([r[...].T for r in refs[:-1]], axis=1)


def _repackT(table_t, d, split):
    v = table_t.shape[1]
    k = 128 // d
    nblk = split // CB
    last = (v + CB - 1) // CB - 1
    specs = [
        pl.BlockSpec(
            (d, CB),
            functools.partial(
                lambda j, i: (0, jnp.minimum(i + j * nblk, last)), j))
        for j in range(k)
    ]
    return pl.pallas_call(
        _repackT_body,
        grid=(nblk,),
        in_specs=specs,
        out_specs=pl.BlockSpec((CB, 128), lambda i: (i, 0)),
        out_shape=jax.ShapeDtypeStruct((split, 128), jnp.float32),
        compiler_params=pltpu.CompilerParams(
            dimension_semantics=("parallel",)),
    )(*([table_t] * k))


BLK = 2048


def _subselect(packed, sel, d):
    out = jnp.zeros((BLK, d), jnp.float32)
    for grp in range(128 // d):
        out += jnp.where(sel == grp, packed[:, grp * d:(grp + 1) * d], 0.0)
    return out


def _mlp_body(num_ref, ea_ref, eb_ref, eg_ref, aid_ref, bid_ref, gid_ref,
              w1n_ref, w1a_ref, w1b_ref, w1g_ref, b1_ref, w2_ref, b2_ref,
              o_ref):
    ea = _subselect(ea_ref[...], aid_ref[...], D_ARTIST)
    eb = _subselect(eb_ref[...], bid_ref[...], D_ALBUM)
    eg = _subselect(eg_ref[...], gid_ref[...], D_GENRE)
    h = jnp.dot(num_ref[...], w1n_ref[...], preferred_element_type=jnp.float32)
    h += jnp.dot(ea, w1a_ref[...], preferred_element_type=jnp.float32)
    h += jnp.dot(eb, w1b_ref[...], preferred_element_type=jnp.float32)
    h += jnp.dot(eg, w1g_ref[...], preferred_element_type=jnp.float32)
    h = jnp.maximum(h + b1_ref[...], 0.0)
    o = jnp.dot(h, w2_ref[...], preferred_element_type=jnp.float32)
    o_ref[...] = jnp.maximum(o + b2_ref[...], 0.0)


def _mlp(num, ea, eb, eg, aid, bid, gid, w1n, w1a, w1b, w1g, b1, w2, b2):
    grid = (B // BLK,)
    full = lambda shape: pl.BlockSpec(shape, lambda i: (0, 0))
    blk2 = lambda shape: pl.BlockSpec(shape, lambda i: (i, 0))
    return pl.pallas_call(
        _mlp_body,
        grid=grid,
        in_specs=[
            blk2((BLK, 9)),
            blk2((BLK, 128)),
            blk2((BLK, 128)),
            blk2((BLK, 128)),
            blk2((BLK, 1)),
            blk2((BLK, 1)),
            blk2((BLK, 1)),
            full((9, H1)),
            full((D_ARTIST, H1)),
            full((D_ALBUM, H1)),
            full((D_GENRE, H1)),
            full((1, H1)),
            full((H1, H2)),
            full((1, H2)),
        ],
        out_specs=blk2((BLK, H2)),
        out_shape=jax.ShapeDtypeStruct((B, H2), jnp.float32),
    )(num, ea, eb, eg, aid, bid, gid, w1n, w1a, w1b, w1g, b1, w2, b2)


def kernel(danceability, energy, loudness, speechiness, acousticness,
           instrumentalness, liveness, valence, tempo,
           artist_id, album_id, genre_id,
           E_artist, E_album, E_genre, W1, b1, W2, b2):
    ap = _repackT(E_artist.T, D_ARTIST, S_ARTIST)
    bp = _repackT(E_album.T, D_ALBUM, S_ALBUM)
    gp = E_genre.reshape(125, 128)
    asel = (artist_id >= S_ARTIST).astype(jnp.int32)
    bsel = album_id // S_ALBUM
    aidx = (artist_id - asel * S_ARTIST).reshape(NW * NWINDOW, WIN)
    bidx = (album_id - bsel * S_ALBUM).reshape(NW * NWINDOW, WIN)
    gidx = (genre_id >> 3).reshape(NW * NWINDOW, WIN)
    ea, eb, eg = _sc_gather(aidx, bidx, gidx, ap, bp, gp)
    num = jnp.stack([danceability, energy, loudness, speechiness, acousticness,
                     instrumentalness, liveness, valence, tempo], axis=1)
    return _mlp(num, ea, eb, eg,
                asel.reshape(B, 1),
                bsel.reshape(B, 1),
                (genre_id & 7).reshape(B, 1),
                W1[:9], W1[9:9 + D_ARTIST],
                W1[9 + D_ARTIST:9 + D_ARTIST + D_ALBUM],
                W1[9 + D_ARTIST + D_ALBUM:],
                b1.reshape(1, H1), W2, b2.reshape(1, H2))


# speedup vs baseline: 2.4768x; 1.4326x over previous
import functools

import jax
import jax.numpy as jnp
from jax import lax
from jax.experimental import pallas as pl
from jax.experimental.pallas import tpu as pltpu
from jax.experimental.pallas import tpu_sc as plsc

B = 16384
D_ARTIST, D_ALBUM, D_GENRE = 64, 32, 16
H1, H2 = 256, 128

NC, NS = 2, 16
NW = NC * NS
BPW = B // NW
WIN = 128
NWINDOW = BPW // WIN

_sc_mesh = plsc.VectorSubcoreMesh(core_axis_name="c", subcore_axis_name="s")


@functools.partial(
    pl.kernel,
    out_type=[
        jax.ShapeDtypeStruct((B, 128), jnp.float32),
        jax.ShapeDtypeStruct((B, 128), jnp.float32),
        jax.ShapeDtypeStruct((B, 128), jnp.float32),
    ],
    mesh=_sc_mesh,
    scratch_types=[
        pltpu.VMEM((1, WIN), jnp.int32),
        pltpu.VMEM((1, WIN), jnp.int32),
        pltpu.VMEM((1, WIN), jnp.int32),
        pltpu.VMEM((WIN, 128), jnp.float32),
        pltpu.VMEM((WIN, 128), jnp.float32),
        pltpu.VMEM((WIN, 128), jnp.float32),
        pltpu.SemaphoreType.DMA,
        pltpu.SemaphoreType.DMA,
        pltpu.SemaphoreType.DMA,
    ],
)
def _sc_gather(aidx_h, bidx_h, gidx_h, ap_h, bp_h, gp_h,
               oa_h, ob_h, og_h,
               ixa, ixb, ixg, ra, rb, rg, sema, semb, semg):
    wid = lax.axis_index("s") * NC + lax.axis_index("c")

    for j in range(NWINDOW):
        r = wid * NWINDOW + j
        pltpu.sync_copy(aidx_h.at[pl.ds(r, 1)], ixa)
        pltpu.sync_copy(bidx_h.at[pl.ds(r, 1)], ixb)
        pltpu.sync_copy(gidx_h.at[pl.ds(r, 1)], ixg)
        ca = pltpu.async_copy(ap_h.at[ixa.at[0]], ra, sema)
        cb = pltpu.async_copy(bp_h.at[ixb.at[0]], rb, semb)
        cg = pltpu.async_copy(gp_h.at[ixg.at[0]], rg, semg)
        ca.wait()
        cb.wait()
        cg.wait()
        base = wid * BPW + j * WIN
        pltpu.sync_copy(ra, oa_h.at[pl.ds(base, WIN)])
        pltpu.sync_copy(rb, ob_h.at[pl.ds(base, WIN)])
        pltpu.sync_copy(rg, og_h.at[pl.ds(base, WIN)])


CB = 2048
S_ARTIST = 501760
S_ALBUM = 251904


def _repackT_body(*refs):
    o_ref = refs[-1]
    v = jnp.concatenate([r[...] for r in refs[:-1]], axis=0)
    o_ref[...] = v.T


def _repackT(table_t, d, split):
    v = table_t.shape[1]
    k = 128 // d
    nblk = split // CB
    last = (v + CB - 1) // CB - 1
    specs = [
        pl.BlockSpec(
            (d, CB),
            functools.partial(
                lambda j, i: (0, jnp.minimum(i + j * nblk, last)), j))
        for j in range(k)
    ]
    return pl.pallas_call(
        _repackT_body,
        grid=(nblk,),
        in_specs=specs,
        out_specs=pl.BlockSpec((CB, 128), lambda i: (i, 0)),
        out_shape=jax.ShapeDtypeStruct((split, 128), jnp.float32),
        compiler_params=pltpu.CompilerParams(
            dimension_semantics=("parallel",)),
    )(*([table_t] * k))


BLK = 2048


def _masked(packed, sel, d):
    grp = lax.broadcasted_iota(jnp.int32, (1, 128), 1) // d
    return jnp.where(sel == grp, packed, 0.0)


def _mlp_body(num_ref, ea_ref, eb_ref, eg_ref, asel_ref, bsel_ref, gsel_ref,
              w1n_ref, w1a_ref, w1b_ref, w1g_ref, b1_ref, w2_ref, b2_ref,
              o_ref):
    ea = _masked(ea_ref[...], asel_ref[...], D_ARTIST)
    eb = _masked(eb_ref[...], bsel_ref[...], D_ALBUM)
    eg = _masked(eg_ref[...], gsel_ref[...], D_GENRE)
    h = jnp.dot(num_ref[...], w1n_ref[...], preferred_element_type=jnp.float32)
    h += jnp.dot(ea, w1a_ref[...], preferred_element_type=jnp.float32)
    h += jnp.dot(eb, w1b_ref[...], preferred_element_type=jnp.float32)
    h += jnp.dot(eg, w1g_ref[...], preferred_element_type=jnp.float32)
    h = jnp.maximum(h + b1_ref[...], 0.0)
    o = jnp.dot(h, w2_ref[...], preferred_element_type=jnp.float32)
    o_ref[...] = jnp.maximum(o + b2_ref[...], 0.0)


def _mlp(num, ea, eb, eg, aid, bid, gid, w1n, w1a, w1b, w1g, b1, w2, b2):
    grid = (B // BLK,)
    full = lambda shape: pl.BlockSpec(shape, lambda i: (0, 0))
    blk2 = lambda shape: pl.BlockSpec(shape, lambda i: (i, 0))
    return pl.pallas_call(
        _mlp_body,
        grid=grid,
        in_specs=[
            blk2((BLK, 9)),
            blk2((BLK, 128)),
            blk2((BLK, 128)),
            blk2((BLK, 128)),
            blk2((BLK, 1)),
            blk2((BLK, 1)),
            blk2((BLK, 1)),
            full((9, H1)),
            full((128, H1)),
            full((128, H1)),
            full((128, H1)),
            full((1, H1)),
            full((H1, H2)),
            full((1, H2)),
        ],
        out_specs=blk2((BLK, H2)),
        out_shape=jax.ShapeDtypeStruct((B, H2), jnp.float32),
    )(num, ea, eb, eg, aid, bid, gid, w1n, w1a, w1b, w1g, b1, w2, b2)


def kernel(danceability, energy, loudness, speechiness, acousticness,
           instrumentalness, liveness, valence, tempo,
           artist_id, album_id, genre_id,
           E_artist, E_album, E_genre, W1, b1, W2, b2):
    ap = _repackT(E_artist.T, D_ARTIST, S_ARTIST)
    bp = _repackT(E_album.T, D_ALBUM, S_ALBUM)
    gp = E_genre.reshape(125, 128)
    asel = (artist_id >= S_ARTIST).astype(jnp.int32)
    bsel = album_id // S_ALBUM
    aidx = (artist_id - asel * S_ARTIST).reshape(NW * NWINDOW, WIN)
    bidx = (album_id - bsel * S_ALBUM).reshape(NW * NWINDOW, WIN)
    gidx = (genre_id >> 3).reshape(NW * NWINDOW, WIN)
    ea, eb, eg = _sc_gather(aidx, bidx, gidx, ap, bp, gp)
    num = jnp.stack([danceability, energy, loudness, speechiness, acousticness,
                     instrumentalness, liveness, valence, tempo], axis=1)
    return _mlp(num, ea, eb, eg,
                asel.reshape(B, 1),
                bsel.reshape(B, 1),
                (genre_id & 7).reshape(B, 1),
                W1[:9],
                jnp.tile(W1[9:9 + D_ARTIST], (2, 1)),
                jnp.tile(W1[9 + D_ARTIST:9 + D_ARTIST + D_ALBUM], (4, 1)),
                jnp.tile(W1[9 + D_ARTIST + D_ALBUM:], (8, 1)),
                b1.reshape(1, H1), W2, b2.reshape(1, H2))


# speedup vs baseline: 3.1544x; 1.2736x over previous
import functools

import jax
import jax.numpy as jnp
from jax import lax
from jax.experimental import pallas as pl
from jax.experimental.pallas import tpu as pltpu
from jax.experimental.pallas import tpu_sc as plsc

B = 16384
D_ARTIST, D_ALBUM, D_GENRE = 64, 32, 16
H1, H2 = 256, 128

NC, NS = 2, 16
NW = NC * NS
BPW = B // NW
WIN = 128
NWINDOW = BPW // WIN

_sc_mesh = plsc.VectorSubcoreMesh(core_axis_name="c", subcore_axis_name="s")


@functools.partial(
    pl.kernel,
    out_type=[
        jax.ShapeDtypeStruct((B, 128), jnp.float32),
        jax.ShapeDtypeStruct((B, 128), jnp.float32),
    ],
    mesh=_sc_mesh,
    scratch_types=[
        pltpu.VMEM((1, WIN), jnp.int32),
        pltpu.VMEM((1, WIN), jnp.int32),
        pltpu.VMEM((WIN, 128), jnp.float32),
        pltpu.VMEM((WIN, 128), jnp.float32),
        pltpu.SemaphoreType.DMA,
        pltpu.SemaphoreType.DMA,
    ],
)
def _sc_gather_bg(bidx_h, gidx_h, bp_h, gp_h, ob_h, og_h,
                  ixb, ixg, rb, rg, semb, semg):
    wid = lax.axis_index("s") * NC + lax.axis_index("c")

    for j in range(NWINDOW):
        r = wid * NWINDOW + j
        pltpu.sync_copy(bidx_h.at[pl.ds(r, 1)], ixb)
        pltpu.sync_copy(gidx_h.at[pl.ds(r, 1)], ixg)
        cb = pltpu.async_copy(bp_h.at[ixb.at[0]], rb, semb)
        cg = pltpu.async_copy(gp_h.at[ixg.at[0]], rg, semg)
        cb.wait()
        cg.wait()
        base = wid * BPW + j * WIN
        pltpu.sync_copy(rb, ob_h.at[pl.ds(base, WIN)])
        pltpu.sync_copy(rg, og_h.at[pl.ds(base, WIN)])


@functools.partial(
    pl.kernel,
    out_type=jax.ShapeDtypeStruct((B, 128), jnp.float32),
    mesh=_sc_mesh,
    scratch_types=[
        pltpu.VMEM((1, WIN), jnp.int32),
        pltpu.VMEM((1, WIN), jnp.int32),
        pltpu.VMEM((WIN, 128), jnp.float32),
        pltpu.VMEM((WIN, 128), jnp.float32),
        pltpu.SemaphoreType.DMA,
        pltpu.SemaphoreType.DMA,
    ],
)
def _sc_gather_a(aidx_h, ap_h, oa_h, ixa0, ixa1, ra0, ra1, sem0, sem1):
    wid = lax.axis_index("s") * NC + lax.axis_index("c")
    ix = (ixa0, ixa1)
    bufs = (ra0, ra1)
    sems = (sem0, sem1)
    copies = [None, None]
    for j in range(NWINDOW):
        k = j % 2
        pltpu.sync_copy(aidx_h.at[pl.ds(wid * NWINDOW + j, 1)], ix[k])
        copies[k] = pltpu.async_copy(ap_h.at[ix[k].at[0]], bufs[k], sems[k])
        if j > 0:
            copies[1 - k].wait()
            pltpu.sync_copy(bufs[1 - k],
                            oa_h.at[pl.ds(wid * BPW + (j - 1) * WIN, WIN)])
    kl = (NWINDOW - 1) % 2
    copies[kl].wait()
    pltpu.sync_copy(bufs[kl],
                    oa_h.at[pl.ds(wid * BPW + (NWINDOW - 1) * WIN, WIN)])


CB = 4096
S_ARTIST = 503808
S_ALBUM = 253952


def _repackT_body(*refs):
    o_ref = refs[-1]
    v = jnp.concatenate([r[...] for r in refs[:-1]], axis=0)
    o_ref[...] = v.T


def _repackT(table_t, d, split):
    v = table_t.shape[1]
    k = 128 // d
    nblk = split // CB
    last = (v + CB - 1) // CB - 1
    specs = [
        pl.BlockSpec(
            (d, CB),
            functools.partial(
                lambda j, i: (0, jnp.minimum(i + j * nblk, last)), j))
        for j in range(k)
    ]
    return pl.pallas_call(
        _repackT_body,
        grid=(nblk,),
        in_specs=specs,
        out_specs=pl.BlockSpec((CB, 128), lambda i: (i, 0)),
        out_shape=jax.ShapeDtypeStruct((split, 128), jnp.float32),
        compiler_params=pltpu.CompilerParams(
            dimension_semantics=("parallel",)),
    )(*([table_t] * k))


BLK = 2048


def _masked(packed, sel, d):
    grp = lax.broadcasted_iota(jnp.int32, (1, 128), 1) // d
    return jnp.where(sel == grp, packed, 0.0)


def _mlp_body(num_ref, ea_ref, eb_ref, eg_ref, asel_ref, bsel_ref, gsel_ref,
              w1n_ref, w1a_ref, w1b_ref, w1g_ref, b1_ref, w2_ref, b2_ref,
              o_ref):
    ea = _masked(ea_ref[...], asel_ref[...], D_ARTIST)
    eb = _masked(eb_ref[...], bsel_ref[...], D_ALBUM)
    eg = _masked(eg_ref[...], gsel_ref[...], D_GENRE)
    h = jnp.dot(num_ref[...], w1n_ref[...], preferred_element_type=jnp.float32)
    h += jnp.dot(ea, w1a_ref[...], preferred_element_type=jnp.float32)
    h += jnp.dot(eb, w1b_ref[...], preferred_element_type=jnp.float32)
    h += jnp.dot(eg, w1g_ref[...], preferred_element_type=jnp.float32)
    h = jnp.maximum(h + b1_ref[...], 0.0)
    o = jnp.dot(h, w2_ref[...], preferred_element_type=jnp.float32)
    o_ref[...] = jnp.maximum(o + b2_ref[...], 0.0)


def _mlp(num, ea, eb, eg, aid, bid, gid, w1n, w1a, w1b, w1g, b1, w2, b2):
    grid = (B // BLK,)
    full = lambda shape: pl.BlockSpec(shape, lambda i: (0, 0))
    blk2 = lambda shape: pl.BlockSpec(shape, lambda i: (i, 0))
    return pl.pallas_call(
        _mlp_body,
        grid=grid,
        in_specs=[
            blk2((BLK, 9)),
            blk2((BLK, 128)),
            blk2((BLK, 128)),
            blk2((BLK, 128)),
            blk2((BLK, 1)),
            blk2((BLK, 1)),
            blk2((BLK, 1)),
            full((9, H1)),
            full((128, H1)),
            full((128, H1)),
            full((128, H1)),
            full((1, H1)),
            full((H1, H2)),
            full((1, H2)),
        ],
        out_specs=blk2((BLK, H2)),
        out_shape=jax.ShapeDtypeStruct((B, H2), jnp.float32),
    )(num, ea, eb, eg, aid, bid, gid, w1n, w1a, w1b, w1g, b1, w2, b2)


def kernel(danceability, energy, loudness, speechiness, acousticness,
           instrumentalness, liveness, valence, tempo,
           artist_id, album_id, genre_id,
           E_artist, E_album, E_genre, W1, b1, W2, b2):
    bp = _repackT(E_album.T, D_ALBUM, S_ALBUM)
    ap = _repackT(E_artist.T, D_ARTIST, S_ARTIST)
    gp = E_genre.reshape(125, 128)
    asel = (artist_id >= S_ARTIST).astype(jnp.int32)
    bsel = album_id // S_ALBUM
    aidx = (artist_id - asel * S_ARTIST).reshape(NW * NWINDOW, WIN)
    bidx = (album_id - bsel * S_ALBUM).reshape(NW * NWINDOW, WIN)
    gidx = (genre_id >> 3).reshape(NW * NWINDOW, WIN)
    eb, eg = _sc_gather_bg(bidx, gidx, bp, gp)
    ea = _sc_gather_a(aidx, ap)
    num = jnp.stack([danceability, energy, loudness, speechiness, acousticness,
                     instrumentalness, liveness, valence, tempo], axis=1)
    return _mlp(num, ea, eb, eg,
                asel.reshape(B, 1),
                bsel.reshape(B, 1),
                (genre_id & 7).reshape(B, 1),
                W1[:9],
                jnp.tile(W1[9:9 + D_ARTIST], (2, 1)),
                jnp.tile(W1[9 + D_ARTIST:9 + D_ARTIST + D_ALBUM], (4, 1)),
                jnp.tile(W1[9 + D_ARTIST + D_ALBUM:], (8, 1)),
                b1.reshape(1, H1), W2, b2.reshape(1, H2))


# speedup vs baseline: 3.4665x; 1.0989x over previous
import functools

import jax
import jax.numpy as jnp
from jax import lax
from jax.experimental import pallas as pl
from jax.experimental.pallas import tpu as pltpu
from jax.experimental.pallas import tpu_sc as plsc

B = 16384
D_ARTIST, D_ALBUM, D_GENRE = 64, 32, 16
H1, H2 = 256, 128

NC, NS = 2, 16
NW = NC * NS
BPW = B // NW
WIN = 128
NWINDOW = BPW // WIN

_sc_mesh = plsc.VectorSubcoreMesh(core_axis_name="c", subcore_axis_name="s")


@functools.partial(
    pl.kernel,
    out_type=[
        jax.ShapeDtypeStruct((B, 128), jnp.float32),
        jax.ShapeDtypeStruct((B, 128), jnp.float32),
    ],
    mesh=_sc_mesh,
    scratch_types=[
        pltpu.VMEM((1, WIN), jnp.int32),
        pltpu.VMEM((1, WIN), jnp.int32),
        pltpu.VMEM((WIN, 128), jnp.float32),
        pltpu.VMEM((WIN, 128), jnp.float32),
        pltpu.SemaphoreType.DMA,
        pltpu.SemaphoreType.DMA,
    ],
)
def _sc_gather_bg(bidx_h, gidx_h, bp_h, gp_h, ob_h, og_h,
                  ixb, ixg, rb, rg, semb, semg):
    wid = lax.axis_index("s") * NC + lax.axis_index("c")

    for j in range(NWINDOW):
        r = wid * NWINDOW + j
        pltpu.sync_copy(bidx_h.at[pl.ds(r, 1)], ixb)
        pltpu.sync_copy(gidx_h.at[pl.ds(r, 1)], ixg)
        cb = pltpu.async_copy(bp_h.at[ixb.at[0]], rb, semb)
        cg = pltpu.async_copy(gp_h.at[ixg.at[0]], rg, semg)
        cb.wait()
        cg.wait()
        base = wid * BPW + j * WIN
        pltpu.sync_copy(rb, ob_h.at[pl.ds(base, WIN)])
        pltpu.sync_copy(rg, og_h.at[pl.ds(base, WIN)])


@functools.partial(
    pl.kernel,
    out_type=jax.ShapeDtypeStruct((B, 128), jnp.float32),
    mesh=_sc_mesh,
    scratch_types=[
        pltpu.VMEM((1, WIN), jnp.int32),
        pltpu.VMEM((1, WIN), jnp.int32),
        pltpu.VMEM((WIN, 128), jnp.float32),
        pltpu.VMEM((WIN, 128), jnp.float32),
        pltpu.SemaphoreType.DMA,
        pltpu.SemaphoreType.DMA,
    ],
)
def _sc_gather_a(aidx_h, ap_h, oa_h, ixa0, ixa1, ra0, ra1, sem0, sem1):
    wid = lax.axis_index("s") * NC + lax.axis_index("c")
    ix = (ixa0, ixa1)
    bufs = (ra0, ra1)
    sems = (sem0, sem1)
    copies = [None, None]
    for j in range(NWINDOW):
        k = j % 2
        pltpu.sync_copy(aidx_h.at[pl.ds(wid * NWINDOW + j, 1)], ix[k])
        copies[k] = pltpu.async_copy(ap_h.at[ix[k].at[0]], bufs[k], sems[k])
        if j > 0:
            copies[1 - k].wait()
            pltpu.sync_copy(bufs[1 - k],
                            oa_h.at[pl.ds(wid * BPW + (j - 1) * WIN, WIN)])
    kl = (NWINDOW - 1) % 2
    copies[kl].wait()
    pltpu.sync_copy(bufs[kl],
                    oa_h.at[pl.ds(wid * BPW + (NWINDOW - 1) * WIN, WIN)])


CB = 8192
S_ARTIST = 507904
S_ALBUM = 253952


def _repackT_body(*refs):
    o_ref = refs[-1]
    v = jnp.concatenate([r[...] for r in refs[:-1]], axis=0)
    o_ref[...] = v.T


def _repackT(table_t, d, split):
    v = table_t.shape[1]
    k = 128 // d
    nblk = split // CB
    last = (v + CB - 1) // CB - 1
    specs = [
        pl.BlockSpec(
            (d, CB),
            functools.partial(
                lambda j, i: (0, jnp.minimum(i + j * nblk, last)), j))
        for j in range(k)
    ]
    return pl.pallas_call(
        _repackT_body,
        grid=(nblk,),
        in_specs=specs,
        out_specs=pl.BlockSpec((CB, 128), lambda i: (i, 0)),
        out_shape=jax.ShapeDtypeStruct((split, 128), jnp.float32),
        compiler_params=pltpu.CompilerParams(
            dimension_semantics=("parallel",)),
    )(*([table_t] * k))


BLK = 2048


def _masked(packed, sel, d):
    grp = lax.broadcasted_iota(jnp.int32, (1, 128), 1) // d
    return jnp.where(sel == grp, packed, 0.0)


def _mlp_body(num_ref, ea_ref, eb_ref, eg_ref, asel_ref, bsel_ref, gsel_ref,
              w1n_ref, w1a_ref, w1b_ref, w1g_ref, b1_ref, w2_ref, b2_ref,
              o_ref):
    ea = _masked(ea_ref[...], asel_ref[...], D_ARTIST)
    eb = _masked(eb_ref[...], bsel_ref[...], D_ALBUM)
    eg = _masked(eg_ref[...], gsel_ref[...], D_GENRE)
    h = jnp.dot(num_ref[...], w1n_ref[...], preferred_element_type=jnp.float32)
    h += jnp.dot(ea, w1a_ref[...], preferred_element_type=jnp.float32)
    h += jnp.dot(eb, w1b_ref[...], preferred_element_type=jnp.float32)
    h += jnp.dot(eg, w1g_ref[...], preferred_element_type=jnp.float32)
    h = jnp.maximum(h + b1_ref[...], 0.0)
    o = jnp.dot(h, w2_ref[...], preferred_element_type=jnp.float32)
    o_ref[...] = jnp.maximum(o + b2_ref[...], 0.0)


def _mlp(num, ea, eb, eg, aid, bid, gid, w1n, w1a, w1b, w1g, b1, w2, b2):
    grid = (B // BLK,)
    full = lambda shape: pl.BlockSpec(shape, lambda i: (0, 0))
    blk2 = lambda shape: pl.BlockSpec(shape, lambda i: (i, 0))
    return pl.pallas_call(
        _mlp_body,
        grid=grid,
        in_specs=[
            blk2((BLK, 9)),
            blk2((BLK, 128)),
            blk2((BLK, 128)),
            blk2((BLK, 128)),
            blk2((BLK, 1)),
            blk2((BLK, 1)),
            blk2((BLK, 1)),
            full((9, H1)),
            full((128, H1)),
            full((128, H1)),
            full((128, H1)),
            full((1, H1)),
            full((H1, H2)),
            full((1, H2)),
        ],
        out_specs=blk2((BLK, H2)),
        out_shape=jax.ShapeDtypeStruct((B, H2), jnp.float32),
    )(num, ea, eb, eg, aid, bid, gid, w1n, w1a, w1b, w1g, b1, w2, b2)


def kernel(danceability, energy, loudness, speechiness, acousticness,
           instrumentalness, liveness, valence, tempo,
           artist_id, album_id, genre_id,
           E_artist, E_album, E_genre, W1, b1, W2, b2):
    asel = (artist_id >= S_ARTIST).astype(jnp.int32)
    bsel = album_id // S_ALBUM
    aidx = (artist_id - asel * S_ARTIST).reshape(NW * NWINDOW, WIN)
    bidx = (album_id - bsel * S_ALBUM).reshape(NW * NWINDOW, WIN)
    gidx = (genre_id >> 3).reshape(NW * NWINDOW, WIN)
    gp = E_genre.reshape(125, 128)
    bp = _repackT(E_album.T, D_ALBUM, S_ALBUM)
    eb, eg = _sc_gather_bg(bidx, gidx, bp, gp)
    ap = _repackT(E_artist.T, D_ARTIST, S_ARTIST)
    ea = _sc_gather_a(aidx, ap)
    num = jnp.stack([danceability, energy, loudness, speechiness, acousticness,
                     instrumentalness, liveness, valence, tempo], axis=1)
    return _mlp(num, ea, eb, eg,
                asel.reshape(B, 1),
                bsel.reshape(B, 1),
                (genre_id & 7).reshape(B, 1),
                W1[:9],
                jnp.tile(W1[9:9 + D_ARTIST], (2, 1)),
                jnp.tile(W1[9 + D_ARTIST:9 + D_ARTIST + D_ALBUM], (4, 1)),
                jnp.tile(W1[9 + D_ARTIST + D_ALBUM:], (8, 1)),
                b1.reshape(1, H1), W2, b2.reshape(1, H2))


# speedup vs baseline: 3.5196x; 1.0153x over previous
import functools

import jax
import jax.numpy as jnp
from jax import lax
from jax.experimental import pallas as pl
from jax.experimental.pallas import tpu as pltpu
from jax.experimental.pallas import tpu_sc as plsc

B = 16384
D_ARTIST, D_ALBUM, D_GENRE = 64, 32, 16
H1, H2 = 256, 128

NC, NS = 2, 16
NW = NC * NS
BPW = B // NW
WIN = 128
NWINDOW = BPW // WIN

_sc_mesh = plsc.VectorSubcoreMesh(core_axis_name="c", subcore_axis_name="s")


@functools.partial(
    pl.kernel,
    out_type=[
        jax.ShapeDtypeStruct((B, 128), jnp.float32),
        jax.ShapeDtypeStruct((B, 128), jnp.float32),
    ],
    mesh=_sc_mesh,
    scratch_types=[
        pltpu.VMEM((1, WIN), jnp.int32),
        pltpu.VMEM((1, WIN), jnp.int32),
        pltpu.VMEM((WIN, 128), jnp.float32),
        pltpu.VMEM((WIN, 128), jnp.float32),
        pltpu.SemaphoreType.DMA,
        pltpu.SemaphoreType.DMA,
    ],
)
def _sc_gather_bg(bidx_h, gidx_h, bp_h, gp_h, ob_h, og_h,
                  ixb, ixg, rb, rg, semb, semg):
    wid = lax.axis_index("s") * NC + lax.axis_index("c")

    for j in range(NWINDOW):
        r = wid * NWINDOW + j
        pltpu.sync_copy(bidx_h.at[pl.ds(r, 1)], ixb)
        pltpu.sync_copy(gidx_h.at[pl.ds(r, 1)], ixg)
        cb = pltpu.async_copy(bp_h.at[ixb.at[0]], rb, semb)
        cg = pltpu.async_copy(gp_h.at[ixg.at[0]], rg, semg)
        cb.wait()
        cg.wait()
        base = wid * BPW + j * WIN
        pltpu.sync_copy(rb, ob_h.at[pl.ds(base, WIN)])
        pltpu.sync_copy(rg, og_h.at[pl.ds(base, WIN)])


@functools.partial(
    pl.kernel,
    out_type=jax.ShapeDtypeStruct((B, 128), jnp.float32),
    mesh=_sc_mesh,
    scratch_types=[
        pltpu.VMEM((1, WIN), jnp.int32),
        pltpu.VMEM((1, WIN), jnp.int32),
        pltpu.VMEM((WIN, 128), jnp.float32),
        pltpu.VMEM((WIN, 128), jnp.float32),
        pltpu.SemaphoreType.DMA,
        pltpu.SemaphoreType.DMA,
    ],
)
def _sc_gather_a(aidx_h, ap_h, oa_h, ixa0, ixa1, ra0, ra1, sem0, sem1):
    wid = lax.axis_index("s") * NC + lax.axis_index("c")
    ix = (ixa0, ixa1)
    bufs = (ra0, ra1)
    sems = (sem0, sem1)
    copies = [None, None]
    for j in range(NWINDOW):
        k = j % 2
        pltpu.sync_copy(aidx_h.at[pl.ds(wid * NWINDOW + j, 1)], ix[k])
        copies[k] = pltpu.async_copy(ap_h.at[ix[k].at[0]], bufs[k], sems[k])
        if j > 0:
            copies[1 - k].wait()
            pltpu.sync_copy(bufs[1 - k],
                            oa_h.at[pl.ds(wid * BPW + (j - 1) * WIN, WIN)])
    kl = (NWINDOW - 1) % 2
    copies[kl].wait()
    pltpu.sync_copy(bufs[kl],
                    oa_h.at[pl.ds(wid * BPW + (NWINDOW - 1) * WIN, WIN)])


CB = 8192
S_ARTIST = 507904
S_ALBUM = 253952


def _repackT_body(*refs):
    o_ref = refs[-1]
    v = jnp.concatenate([r[...] for r in refs[:-1]], axis=0)
    o_ref[...] = v.T


def _repackT(table_t, d, split, cb):
    v = table_t.shape[1]
    k = 128 // d
    nblk = split // cb
    last = (v + cb - 1) // cb - 1
    specs = [
        pl.BlockSpec(
            (d, cb),
            functools.partial(
                lambda j, i: (0, jnp.minimum(i + j * nblk, last)), j))
        for j in range(k)
    ]
    return pl.pallas_call(
        _repackT_body,
        grid=(nblk,),
        in_specs=specs,
        out_specs=pl.BlockSpec((cb, 128), lambda i: (i, 0)),
        out_shape=jax.ShapeDtypeStruct((split, 128), jnp.float32),
        compiler_params=pltpu.CompilerParams(
            dimension_semantics=("parallel",)),
    )(*([table_t] * k))


BLK = 2048


def _masked(packed, sel, d):
    grp = lax.broadcasted_iota(jnp.int32, (1, 128), 1) // d
    return jnp.where(sel == grp, packed, 0.0)


def _mlp_body(num_ref, ea_ref, eb_ref, eg_ref, asel_ref, bsel_ref, gsel_ref,
              w1n_ref, w1a_ref, w1b_ref, w1g_ref, b1_ref, w2_ref, b2_ref,
              o_ref):
    ea = _masked(ea_ref[...], asel_ref[...], D_ARTIST)
    eb = _masked(eb_ref[...], bsel_ref[...], D_ALBUM)
    eg = _masked(eg_ref[...], gsel_ref[...], D_GENRE)
    h = jnp.dot(num_ref[...], w1n_ref[...], preferred_element_type=jnp.float32)
    h += jnp.dot(ea, w1a_ref[...], preferred_element_type=jnp.float32)
    h += jnp.dot(eb, w1b_ref[...], preferred_element_type=jnp.float32)
    h += jnp.dot(eg, w1g_ref[...], preferred_element_type=jnp.float32)
    h = jnp.maximum(h + b1_ref[...], 0.0)
    o = jnp.dot(h, w2_ref[...], preferred_element_type=jnp.float32)
    o_ref[...] = jnp.maximum(o + b2_ref[...], 0.0)


def _mlp(num, ea, eb, eg, aid, bid, gid, w1n, w1a, w1b, w1g, b1, w2, b2):
    grid = (B // BLK,)
    full = lambda shape: pl.BlockSpec(shape, lambda i: (0, 0))
    blk2 = lambda shape: pl.BlockSpec(shape, lambda i: (i, 0))
    return pl.pallas_call(
        _mlp_body,
        grid=grid,
        in_specs=[
            blk2((BLK, 9)),
            blk2((BLK, 128)),
            blk2((BLK, 128)),
            blk2((BLK, 128)),
            blk2((BLK, 1)),
            blk2((BLK, 1)),
            blk2((BLK, 1)),
            full((9, H1)),
            full((128, H1)),
            full((128, H1)),
            full((128, H1)),
            full((1, H1)),
            full((H1, H2)),
            full((1, H2)),
        ],
        out_specs=blk2((BLK, H2)),
        out_shape=jax.ShapeDtypeStruct((B, H2), jnp.float32),
    )(num, ea, eb, eg, aid, bid, gid, w1n, w1a, w1b, w1g, b1, w2, b2)


def kernel(danceability, energy, loudness, speechiness, acousticness,
           instrumentalness, liveness, valence, tempo,
           artist_id, album_id, genre_id,
           E_artist, E_album, E_genre, W1, b1, W2, b2):
    asel = (artist_id >= S_ARTIST).astype(jnp.int32)
    bsel = album_id // S_ALBUM
    aidx = (artist_id - asel * S_ARTIST).reshape(NW * NWINDOW, WIN)
    bidx = (album_id - bsel * S_ALBUM).reshape(NW * NWINDOW, WIN)
    gidx = (genre_id >> 3).reshape(NW * NWINDOW, WIN)
    gp = E_genre.reshape(125, 128)
    bp = _repackT(E_album.T, D_ALBUM, S_ALBUM, 8192)
    eb, eg = _sc_gather_bg(bidx, gidx, bp, gp)
    ap = _repackT(E_artist.T, D_ARTIST, S_ARTIST, 16384)
    ea = _sc_gather_a(aidx, ap)
    num = jnp.stack([danceability, energy, loudness, speechiness, acousticness,
                     instrumentalness, liveness, valence, tempo], axis=1)
    return _mlp(num, ea, eb, eg,
                asel.reshape(B, 1),
                bsel.reshape(B, 1),
                (genre_id & 7).reshape(B, 1),
                W1[:9],
                jnp.tile(W1[9:9 + D_ARTIST], (2, 1)),
                jnp.tile(W1[9 + D_ARTIST:9 + D_ARTIST + D_ALBUM], (4, 1)),
                jnp.tile(W1[9 + D_ARTIST + D_ALBUM:], (8, 1)),
                b1.reshape(1, H1), W2, b2.reshape(1, H2))
